# baseline (speedup 1.0000x reference)
import functools

import jax
import jax.numpy as jnp
from jax import lax
from jax.experimental import pallas as pl
from jax.experimental.pallas import tpu as pltpu
from jax.experimental.pallas import tpu_sc as plsc

NCORES = 2
NSUB = 16
NTILES = NCORES * NSUB
CHUNK = 128
NBUF = 2
NIDX = 4
ZROWS = 64


def _round_up(a, b):
  return (a + b - 1) // b * b


def _sc_aggregate(x, edges4):
  n_nodes, d = x.shape
  chunks_per_tile = edges4.shape[1]
  acc_rows = _round_up(n_nodes + 1, NSUB * ZROWS)
  zdmas_per_tile = acc_rows // NSUB // ZROWS
  rows_per_tile = acc_rows // NSUB

  mesh = plsc.VectorSubcoreMesh(core_axis_name="c", subcore_axis_name="s")

  @functools.partial(
      pl.kernel,
      out_type=(jax.ShapeDtypeStruct((NCORES, acc_rows, d), jnp.float32),
                jax.ShapeDtypeStruct((NCORES, acc_rows), jnp.float32)),
      mesh=mesh,
      scratch_types=[
          pltpu.VMEM((NIDX, 2, CHUNK), jnp.int32),
          pltpu.VMEM((NBUF, CHUNK, d), jnp.float32),
          pltpu.VMEM((acc_rows // NSUB,), jnp.float32),
          pltpu.VMEM((CHUNK,), jnp.float32),
          pltpu.VMEM_SHARED((acc_rows, d), jnp.float32),
          pltpu.VMEM_SHARED((acc_rows,), jnp.float32),
          pltpu.SemaphoreType.DMA((NIDX,)),
          pltpu.SemaphoreType.DMA((NBUF,)),
          pltpu.SemaphoreType.DMA((NBUF,)),
          pltpu.SemaphoreType.DMA((NBUF,)),
      ],
  )
  def k(x_hbm, edges_hbm, out_hbm, cnt_hbm, idx_v, rows_v,
        zero1_v, ones_v, acc_sh, cnt_sh, sem_i, sem_g, sem_s, sem_c):
    cid = lax.axis_index("c")
    sid = lax.axis_index("s")
    wid = sid * NCORES + cid

    def zrow(i, _):
      def zcol(kk, _):
        rows_v[0, i, pl.ds(kk * 16, 16)] = jnp.zeros((16,), jnp.float32)
        return 0
      return lax.fori_loop(0, d // 16, zcol, 0)
    lax.fori_loop(0, ZROWS, zrow, 0)

    def z1d(i, _):
      zero1_v[pl.ds(i * 16, 16)] = jnp.zeros((16,), jnp.float32)
      return 0
    lax.fori_loop(0, rows_per_tile // 16, z1d, 0)

    def o1d(i, _):
      ones_v[pl.ds(i * 16, 16)] = jnp.ones((16,), jnp.float32)
      return 0
    lax.fori_loop(0, CHUNK // 16, o1d, 0)

    def zdma(b, _):
      base = sid * rows_per_tile + b * ZROWS
      pltpu.sync_copy(rows_v.at[0, pl.ds(0, ZROWS)],
                      acc_sh.at[pl.ds(base, ZROWS)])
      return 0
    lax.fori_loop(0, zdmas_per_tile, zdma, 0)
    pltpu.sync_copy(zero1_v, cnt_sh.at[pl.ds(sid * rows_per_tile,
                                             rows_per_tile)])

    plsc.subcore_barrier()

    n_ch = chunks_per_tile

    for q in range(3):
      pltpu.async_copy(edges_hbm.at[wid, q], idx_v.at[q], sem_i.at[q])
    pltpu.make_async_copy(edges_hbm.at[0, 0], idx_v.at[0],
                          sem_i.at[0]).wait()
    pltpu.async_copy(x_hbm.at[idx_v.at[0, 0]], rows_v.at[0], sem_g.at[0])

    def outer(g, _):
      for b in range(NBUF):
        i = g * NBUF + b
        b2 = (b + 1) % NBUF
        pltpu.make_async_copy(x_hbm.at[pl.ds(0, CHUNK)], rows_v.at[b],
                              sem_g.at[b]).wait()
        q0 = lax.rem(i, NIDX)
        pltpu.async_copy(rows_v.at[b], acc_sh.at[idx_v.at[q0, 1]],
                         sem_s.at[b], add=True)
        pltpu.async_copy(ones_v, cnt_sh.at[idx_v.at[q0, 1]], sem_c.at[b],
                         add=True)

        @pl.when(i < n_ch - 1)
        def _():
          @pl.when(i >= 1)
          def _():
            pltpu.make_async_copy(rows_v.at[b2], acc_sh.at[pl.ds(0, CHUNK)],
                                  sem_s.at[b2]).wait()
            pltpu.make_async_copy(ones_v, cnt_sh.at[pl.ds(0, CHUNK)],
                                  sem_c.at[b2]).wait()
          q1 = lax.rem(i + 1, NIDX)
          pltpu.make_async_copy(edges_hbm.at[0, 0], idx_v.at[q1],
                                sem_i.at[q1]).wait()
          pltpu.async_copy(x_hbm.at[idx_v.at[q1, 0]], rows_v.at[b2],
                           sem_g.at[b2])

          @pl.when(i + 3 < n_ch)
          def _():
            q3 = lax.rem(i + 3, NIDX)
            pltpu.async_copy(edges_hbm.at[wid, i + 3], idx_v.at[q3],
                             sem_i.at[q3])
      return 0
    lax.fori_loop(0, n_ch // NBUF, outer, 0)

    for b in range(NBUF):
      pltpu.make_async_copy(rows_v.at[b], acc_sh.at[pl.ds(0, CHUNK)],
                            sem_s.at[b]).wait()
      pltpu.make_async_copy(ones_v, cnt_sh.at[pl.ds(0, CHUNK)],
                            sem_c.at[b]).wait()

    plsc.subcore_barrier()

    pltpu.sync_copy(
        acc_sh.at[pl.ds(sid * rows_per_tile, rows_per_tile)],
        out_hbm.at[cid, pl.ds(sid * rows_per_tile, rows_per_tile)])
    pltpu.sync_copy(
        cnt_sh.at[pl.ds(sid * rows_per_tile, rows_per_tile)],
        cnt_hbm.at[cid, pl.ds(sid * rows_per_tile, rows_per_tile)])

  return k(x, edges4)


def _tc_body(part_ref, cnt_ref, x_ref, wl_ref, bl_ref, wr_ref, o_ref):
  s = part_ref[0] + part_ref[1]
  c = cnt_ref[0] + cnt_ref[1]
  mean = s / jnp.maximum(c, 1.0)
  z = (lax.dot_general(mean, wl_ref[...], (((1,), (1,)), ((), ())),
                       preferred_element_type=jnp.float32)
       + bl_ref[...]
       + lax.dot_general(x_ref[...], wr_ref[...], (((1,), (1,)), ((), ())),
                         preferred_element_type=jnp.float32))
  m = jnp.max(z, axis=1, keepdims=True)
  e = z - m
  lse = jnp.log(jnp.sum(jnp.exp(e), axis=1, keepdims=True))
  o_ref[...] = e - lse


def _tc_finish(part, cnt, x, w_l, b_l, w_r):
  n, d_in = x.shape
  d_out = w_l.shape[0]
  blk = 400
  return pl.pallas_call(
      _tc_body,
      grid=(n // blk,),
      in_specs=[
          pl.BlockSpec((NCORES, blk, d_in), lambda i: (0, i, 0)),
          pl.BlockSpec((NCORES, blk, 1), lambda i: (0, i, 0)),
          pl.BlockSpec((blk, d_in), lambda i: (i, 0)),
          pl.BlockSpec((d_out, d_in), lambda i: (0, 0)),
          pl.BlockSpec((1, d_out), lambda i: (0, 0)),
          pl.BlockSpec((d_out, d_in), lambda i: (0, 0)),
      ],
      out_specs=pl.BlockSpec((blk, d_out), lambda i: (i, 0)),
      out_shape=jax.ShapeDtypeStruct((n, d_out), jnp.float32),
  )(part, cnt, x, w_l, b_l.reshape(1, d_out), w_r)


@jax.jit
def kernel(x, edge_index, W_l, b_l, W_r):
  n, d_in = x.shape
  e = edge_index.shape[1]
  chunks_per_tile = _round_up(-(-e // (NTILES * CHUNK)), NBUF)
  e_pad = NTILES * chunks_per_tile * CHUNK
  src = edge_index[0].astype(jnp.int32)
  dst = edge_index[1].astype(jnp.int32)
  src = jnp.concatenate([src, jnp.zeros((e_pad - e,), jnp.int32)])
  dst = jnp.concatenate([dst, jnp.full((e_pad - e,), n, jnp.int32)])
  edges4 = jnp.stack(
      [src.reshape(NTILES, chunks_per_tile, CHUNK),
       dst.reshape(NTILES, chunks_per_tile, CHUNK)], axis=2)
  part, cnt = _sc_aggregate(x, edges4)
  return _tc_finish(part, cnt[:, :, None], x, W_l, b_l, W_r)


# speedup vs baseline: 2.7954x; 2.7954x over previous
import functools

import jax
import jax.numpy as jnp
from jax import lax
from jax.experimental import pallas as pl
from jax.experimental.pallas import tpu as pltpu
from jax.experimental.pallas import tpu_sc as plsc

NCORES = 2
NSUB = 16
NTILES = NCORES * NSUB
CHUNK = 128
NBUF = 2
NIDX = 4
ZROWS = 64


def _round_up(a, b):
  return (a + b - 1) // b * b


def _sc_aggregate(x, edges):
  n_nodes, d = x.shape
  n_chunks = edges.shape[1]
  acc_rows = _round_up(n_nodes + 1, NSUB * ZROWS)
  zdmas_per_tile = acc_rows // NSUB // ZROWS
  rows_per_tile = acc_rows // NSUB

  mesh = plsc.VectorSubcoreMesh(core_axis_name="c", subcore_axis_name="s")

  @functools.partial(
      pl.kernel,
      out_type=(jax.ShapeDtypeStruct((NCORES, acc_rows, d), jnp.float32),
                jax.ShapeDtypeStruct((NCORES, acc_rows), jnp.float32)),
      mesh=mesh,
      scratch_types=[
          pltpu.VMEM((NIDX, 2, CHUNK), jnp.int32),
          pltpu.VMEM((NBUF, CHUNK, d), jnp.float32),
          pltpu.VMEM((acc_rows // NSUB,), jnp.float32),
          pltpu.VMEM((CHUNK,), jnp.float32),
          pltpu.VMEM_SHARED((acc_rows, d), jnp.float32),
          pltpu.VMEM_SHARED((acc_rows,), jnp.float32),
          pltpu.SemaphoreType.DMA((NIDX,)),
          pltpu.SemaphoreType.DMA((NBUF,)),
          pltpu.SemaphoreType.DMA((NBUF,)),
          pltpu.SemaphoreType.DMA((NBUF,)),
      ],
  )
  def k(x_hbm, edges_hbm, out_hbm, cnt_hbm, idx_v, rows_v,
        zero1_v, ones_v, acc_sh, cnt_sh, sem_i, sem_g, sem_s, sem_c):
    cid = lax.axis_index("c")
    sid = lax.axis_index("s")
    wid = sid * NCORES + cid
    n_ch = (n_chunks - wid + NTILES - 1) // NTILES

    def zrow(i, _):
      def zcol(kk, _):
        rows_v[0, i, pl.ds(kk * 16, 16)] = jnp.zeros((16,), jnp.float32)
        return 0
      return lax.fori_loop(0, d // 16, zcol, 0)
    lax.fori_loop(0, ZROWS, zrow, 0)

    def z1d(i, _):
      zero1_v[pl.ds(i * 16, 16)] = jnp.zeros((16,), jnp.float32)
      return 0
    lax.fori_loop(0, rows_per_tile // 16, z1d, 0)

    def o1d(i, _):
      ones_v[pl.ds(i * 16, 16)] = jnp.ones((16,), jnp.float32)
      return 0
    lax.fori_loop(0, CHUNK // 16, o1d, 0)

    def zdma(b, _):
      base = sid * rows_per_tile + b * ZROWS
      pltpu.sync_copy(rows_v.at[0, pl.ds(0, ZROWS)],
                      acc_sh.at[pl.ds(base, ZROWS)])
      return 0
    lax.fori_loop(0, zdmas_per_tile, zdma, 0)
    pltpu.sync_copy(zero1_v, cnt_sh.at[pl.ds(sid * rows_per_tile,
                                             rows_per_tile)])

    plsc.subcore_barrier()

    def idx_load(i, q):
      pltpu.async_copy(edges_hbm.at[:, wid + i * NTILES], idx_v.at[q],
                       sem_i.at[q])

    for q in range(3):
      idx_load(q, q)
    pltpu.make_async_copy(edges_hbm.at[:, 0], idx_v.at[0],
                          sem_i.at[0]).wait()
    pltpu.async_copy(x_hbm.at[idx_v.at[0, 0]], rows_v.at[0], sem_g.at[0])

    def body(i, _):
      b = lax.rem(i, NBUF)
      b2 = lax.rem(i + 1, NBUF)
      pltpu.make_async_copy(x_hbm.at[pl.ds(0, CHUNK)], rows_v.at[b],
                            sem_g.at[b]).wait()
      q0 = lax.rem(i, NIDX)
      pltpu.async_copy(rows_v.at[b], acc_sh.at[idx_v.at[q0, 1]],
                       sem_s.at[b], add=True)
      pltpu.async_copy(ones_v, cnt_sh.at[idx_v.at[q0, 1]], sem_c.at[b],
                       add=True)

      @pl.when(i < n_ch - 1)
      def _():
        @pl.when(i >= 1)
        def _():
          pltpu.make_async_copy(rows_v.at[b2], acc_sh.at[pl.ds(0, CHUNK)],
                                sem_s.at[b2]).wait()
          pltpu.make_async_copy(ones_v, cnt_sh.at[pl.ds(0, CHUNK)],
                                sem_c.at[b2]).wait()
        q1 = lax.rem(i + 1, NIDX)
        pltpu.make_async_copy(edges_hbm.at[:, 0], idx_v.at[q1],
                              sem_i.at[q1]).wait()
        pltpu.async_copy(x_hbm.at[idx_v.at[q1, 0]], rows_v.at[b2],
                         sem_g.at[b2])

        @pl.when(i + 3 < n_ch)
        def _():
          idx_load(i + 3, lax.rem(i + 3, NIDX))
      return 0
    lax.fori_loop(0, n_ch, body, 0)

    for b in range(NBUF):
      pltpu.make_async_copy(rows_v.at[b], acc_sh.at[pl.ds(0, CHUNK)],
                            sem_s.at[b]).wait()
      pltpu.make_async_copy(ones_v, cnt_sh.at[pl.ds(0, CHUNK)],
                            sem_c.at[b]).wait()

    plsc.subcore_barrier()

    pltpu.sync_copy(
        acc_sh.at[pl.ds(sid * rows_per_tile, rows_per_tile)],
        out_hbm.at[cid, pl.ds(sid * rows_per_tile, rows_per_tile)])
    pltpu.sync_copy(
        cnt_sh.at[pl.ds(sid * rows_per_tile, rows_per_tile)],
        cnt_hbm.at[cid, pl.ds(sid * rows_per_tile, rows_per_tile)])

  return k(x, edges)


def _tc_body(part_ref, cnt_ref, x_ref, wl_ref, bl_ref, wr_ref, o_ref):
  s = part_ref[0] + part_ref[1]
  c = cnt_ref[0] + cnt_ref[1]
  mean = s / jnp.maximum(c, 1.0)
  z = (lax.dot_general(mean, wl_ref[...], (((1,), (1,)), ((), ())),
                       preferred_element_type=jnp.float32)
       + bl_ref[...]
       + lax.dot_general(x_ref[...], wr_ref[...], (((1,), (1,)), ((), ())),
                         preferred_element_type=jnp.float32))
  m = jnp.max(z, axis=1, keepdims=True)
  e = z - m
  lse = jnp.log(jnp.sum(jnp.exp(e), axis=1, keepdims=True))
  o_ref[...] = e - lse


def _tc_finish(part, cnt, x, w_l, b_l, w_r):
  n, d_in = x.shape
  d_out = w_l.shape[0]
  blk = 2000
  return pl.pallas_call(
      _tc_body,
      grid=(n // blk,),
      in_specs=[
          pl.BlockSpec((NCORES, blk, d_in), lambda i: (0, i, 0)),
          pl.BlockSpec((NCORES, blk, 1), lambda i: (0, i, 0)),
          pl.BlockSpec((blk, d_in), lambda i: (i, 0)),
          pl.BlockSpec((d_out, d_in), lambda i: (0, 0)),
          pl.BlockSpec((1, d_out), lambda i: (0, 0)),
          pl.BlockSpec((d_out, d_in), lambda i: (0, 0)),
      ],
      out_specs=pl.BlockSpec((blk, d_out), lambda i: (i, 0)),
      out_shape=jax.ShapeDtypeStruct((n, d_out), jnp.float32),
  )(part, cnt, x, w_l, b_l.reshape(1, d_out), w_r)


@jax.jit
def kernel(x, edge_index, W_l, b_l, W_r):
  e = edge_index.shape[1]
  assert e % CHUNK == 0
  edges = edge_index.astype(jnp.int32).reshape(2, e // CHUNK, CHUNK)
  part, cnt = _sc_aggregate(x, edges)
  return _tc_finish(part, cnt[:, :, None], x, W_l, b_l, W_r)
